# baseline (speedup 1.0000x reference)
import jax
import jax.numpy as jnp
from jax.experimental import pallas as pl
from jax.experimental.pallas import tpu as pltpu
from jax.experimental.pallas import tpu_sc as plsc

T, H, E, F, TOP_K = 32, 2048, 16, 1408, 2
HH = H // 2
FH = F // 2


def _scale_from_combine(combine, e):
    iota = jax.lax.broadcasted_iota(jnp.int32, (T, E), 1)
    return jnp.sum(jnp.where(iota == e, combine, 0.0), axis=1, keepdims=True)


def _router_combine(router_logits):
    info = plsc.get_sparse_core_info()
    nw = info.num_cores * info.num_subcores
    tpw = -(-T // nw)
    mesh = plsc.VectorSubcoreMesh(core_axis_name="c", subcore_axis_name="s")

    def body(logits_hbm, out_hbm, lv, ov):
        wid = jax.lax.axis_index("s") * info.num_cores + jax.lax.axis_index("c")
        for i in range(tpw):
            t = wid * tpw + i

            @pl.when(t < T)
            def _():
                pltpu.sync_copy(logits_hbm.at[t], lv)
                l = lv[...]
                iota = jax.lax.iota(jnp.int32, E)
                m1 = jnp.max(l)
                idx1 = jnp.min(jnp.where(l >= m1, iota, E))
                masked = jnp.where(iota == idx1, -jnp.inf, l)
                m2 = jnp.max(masked)
                idx2 = jnp.min(jnp.where(masked >= m2, iota, E))
                e2 = jnp.max(jnp.exp(masked - m1))
                y = 0.5 * (1.0 + e2)
                r = 2.8235294 - 1.8823529 * y
                r = r * (2.0 - y * r)
                r = r * (2.0 - y * r)
                r = r * (2.0 - y * r)
                w2 = e2 * (0.5 * r)
                w1 = 1.0 - w2
                ov[...] = (jnp.where(iota == idx1, w1, 0.0)
                           + jnp.where(iota == idx2, w2, 0.0))
                pltpu.sync_copy(ov, out_hbm.at[t])

    return pl.kernel(
        body,
        out_type=jax.ShapeDtypeStruct((T, E), jnp.float32),
        mesh=mesh,
        scratch_types=[
            pltpu.VMEM((E,), jnp.float32),
            pltpu.VMEM((E,), jnp.float32),
        ],
        compiler_params=pltpu.CompilerParams(needs_layout_passes=False),
    )(router_logits)


def _moe_kernel(x_ref, combine_ref, wg_hbm, wu_hbm, wd_hbm, out_ref,
                a_buf, d_buf, a_sem, d_sem):
    def issue_a(e, j, slot):
        w = wg_hbm if j < 2 else wu_hbm
        h0 = (j % 2) * HH
        pltpu.make_async_copy(
            w.at[e, pl.ds(h0, HH), :], a_buf.at[slot], a_sem.at[slot]).start()

    def issue_d(e, j, slot):
        pltpu.make_async_copy(
            wd_hbm.at[e, pl.ds(j * FH, FH), :],
            d_buf.at[slot], d_sem.at[slot]).start()

    for c in range(6):
        issue_a(c // 4, c % 4, c)
    for c in range(3):
        issue_d(c // 2, c % 2, c)

    x = x_ref[...]
    x0 = x[:, :HH]
    x1 = x[:, HH:]
    combine = combine_ref[...]

    out_ref[...] = jnp.zeros((T, H), dtype=jnp.float32)

    def body(e, carry):
        def consume_a(j, xh):
            sa = jax.lax.rem(4 * e + j, 6)
            pltpu.make_async_copy(
                a_buf.at[sa], a_buf.at[sa], a_sem.at[sa]).wait()
            r = jnp.dot(xh, a_buf[sa], preferred_element_type=jnp.float32)
            e_i = e + 1 if j < 2 else e + 2
            j_i = (j + 2) % 4

            @pl.when(e_i < E)
            def _():
                issue_a(e_i, j_i, sa)
            return r

        g = consume_a(0, x0)
        g += consume_a(1, x1)
        u = consume_a(2, x0)
        u += consume_a(3, x1)

        scale = _scale_from_combine(combine, e)
        act = scale * ((g * jax.lax.logistic(g)) * u)

        def consume_d(j):
            sd = jax.lax.rem(2 * e + j, 3)
            pltpu.make_async_copy(
                d_buf.at[sd], d_buf.at[sd], d_sem.at[sd]).wait()
            out_ref[...] += jnp.dot(act[:, j * FH:(j + 1) * FH], d_buf[sd],
                                    preferred_element_type=jnp.float32)
            e_i = e + 1 if j == 0 else e + 2
            j_i = 1 - j

            @pl.when(e_i < E)
            def _():
                issue_d(e_i, j_i, sd)

        consume_d(0)
        consume_d(1)
        return carry

    jax.lax.fori_loop(0, E, body, 0)


def kernel(hidden_states, router_logits, W_gate, W_up, W_down):
    combine = _router_combine(router_logits)
    return pl.pallas_call(
        _moe_kernel,
        in_specs=[
            pl.BlockSpec(memory_space=pltpu.VMEM),
            pl.BlockSpec(memory_space=pltpu.VMEM),
            pl.BlockSpec(memory_space=pl.ANY),
            pl.BlockSpec(memory_space=pl.ANY),
            pl.BlockSpec(memory_space=pl.ANY),
        ],
        out_specs=pl.BlockSpec(memory_space=pltpu.VMEM),
        out_shape=jax.ShapeDtypeStruct((T, H), jnp.float32),
        scratch_shapes=[
            pltpu.VMEM((6, HH, F), jnp.float32),
            pltpu.VMEM((3, FH, H), jnp.float32),
            pltpu.SemaphoreType.DMA((6,)),
            pltpu.SemaphoreType.DMA((3,)),
        ],
        compiler_params=pltpu.CompilerParams(
            vmem_limit_bytes=60 * 1024 * 1024,
        ),
    )(hidden_states, combine, W_gate, W_up, W_down)


# speedup vs baseline: 1.0118x; 1.0118x over previous
import jax
import jax.numpy as jnp
from jax.experimental import pallas as pl
from jax.experimental.pallas import tpu as pltpu
from jax.experimental.pallas import tpu_sc as plsc

T, H, E, F, TOP_K = 32, 2048, 16, 1408, 2
HH = H // 2
FH = F // 2


def _scale_from_combine(combine, e):
    iota = jax.lax.broadcasted_iota(jnp.int32, (T, E), 1)
    return jnp.sum(jnp.where(iota == e, combine, 0.0), axis=1, keepdims=True)


def _router_combine(router_logits):
    info = plsc.get_sparse_core_info()
    nw = info.num_cores * info.num_subcores
    tpw = -(-T // nw)
    mesh = plsc.VectorSubcoreMesh(core_axis_name="c", subcore_axis_name="s")

    def body(logits_hbm, out_hbm, lv, ov):
        wid = jax.lax.axis_index("s") * info.num_cores + jax.lax.axis_index("c")
        for i in range(tpw):
            t = wid * tpw + i

            @pl.when(t < T)
            def _():
                pltpu.sync_copy(logits_hbm.at[t], lv)
                l = lv[...]
                iota = jax.lax.iota(jnp.int32, E)
                m1 = jnp.max(l)
                idx1 = jnp.min(jnp.where(l >= m1, iota, E))
                masked = jnp.where(iota == idx1, -jnp.inf, l)
                m2 = jnp.max(masked)
                idx2 = jnp.min(jnp.where(masked >= m2, iota, E))
                e2 = jnp.max(jnp.exp(masked - m1))
                y = 0.5 * (1.0 + e2)
                r = 2.8235294 - 1.8823529 * y
                r = r * (2.0 - y * r)
                r = r * (2.0 - y * r)
                r = r * (2.0 - y * r)
                w2 = e2 * (0.5 * r)
                w1 = 1.0 - w2
                ov[...] = (jnp.where(iota == idx1, w1, 0.0)
                           + jnp.where(iota == idx2, w2, 0.0))
                pltpu.sync_copy(ov, out_hbm.at[t])

    return pl.kernel(
        body,
        out_type=jax.ShapeDtypeStruct((T, E), jnp.float32),
        mesh=mesh,
        scratch_types=[
            pltpu.VMEM((E,), jnp.float32),
            pltpu.VMEM((E,), jnp.float32),
        ],
        compiler_params=pltpu.CompilerParams(needs_layout_passes=False),
    )(router_logits)


def _moe_kernel(x_ref, combine_ref, wg_hbm, wu_hbm, wd_hbm, out_ref,
                a_buf, d_buf, a_sem, d_sem):
    def issue_a(e, j, slot):
        w = wg_hbm if j < 2 else wu_hbm
        h0 = (j % 2) * HH
        pltpu.make_async_copy(
            w.at[e, pl.ds(h0, HH), :], a_buf.at[slot], a_sem.at[slot]).start()

    def issue_d(e, j, slot):
        pltpu.make_async_copy(
            wd_hbm.at[e, pl.ds(j * FH, FH), :],
            d_buf.at[slot], d_sem.at[slot]).start()

    for c in range(7):
        issue_a(c // 4, c % 4, c)
    for c in range(3):
        issue_d(c // 2, c % 2, c)

    x = x_ref[...]
    x0 = x[:, :HH]
    x1 = x[:, HH:]
    combine = combine_ref[...]

    out_ref[...] = jnp.zeros((T, H), dtype=jnp.float32)

    def body(e, carry):
        def consume_a(j, xh):
            sa = jax.lax.rem(4 * e + j, 7)
            pltpu.make_async_copy(
                a_buf.at[sa], a_buf.at[sa], a_sem.at[sa]).wait()
            r = jnp.dot(xh, a_buf[sa], preferred_element_type=jnp.float32)
            e_i = e + 1 if j == 0 else e + 2
            j_i = (j + 3) % 4

            @pl.when(e_i < E)
            def _():
                issue_a(e_i, j_i, sa)
            return r

        g = consume_a(0, x0)
        g += consume_a(1, x1)
        u = consume_a(2, x0)
        u += consume_a(3, x1)

        scale = _scale_from_combine(combine, e)
        act = scale * ((g * jax.lax.logistic(g)) * u)

        def consume_d(j):
            sd = jax.lax.rem(2 * e + j, 3)
            pltpu.make_async_copy(
                d_buf.at[sd], d_buf.at[sd], d_sem.at[sd]).wait()
            out_ref[...] += jnp.dot(act[:, j * FH:(j + 1) * FH], d_buf[sd],
                                    preferred_element_type=jnp.float32)
            e_i = e + 1 if j == 0 else e + 2
            j_i = 1 - j

            @pl.when(e_i < E)
            def _():
                issue_d(e_i, j_i, sd)

        consume_d(0)
        consume_d(1)
        return carry

    jax.lax.fori_loop(0, E, body, 0)


def kernel(hidden_states, router_logits, W_gate, W_up, W_down):
    combine = _router_combine(router_logits)
    return pl.pallas_call(
        _moe_kernel,
        in_specs=[
            pl.BlockSpec(memory_space=pltpu.VMEM),
            pl.BlockSpec(memory_space=pltpu.VMEM),
            pl.BlockSpec(memory_space=pl.ANY),
            pl.BlockSpec(memory_space=pl.ANY),
            pl.BlockSpec(memory_space=pl.ANY),
        ],
        out_specs=pl.BlockSpec(memory_space=pltpu.VMEM),
        out_shape=jax.ShapeDtypeStruct((T, H), jnp.float32),
        scratch_shapes=[
            pltpu.VMEM((7, HH, F), jnp.float32),
            pltpu.VMEM((3, FH, H), jnp.float32),
            pltpu.SemaphoreType.DMA((7,)),
            pltpu.SemaphoreType.DMA((3,)),
        ],
        compiler_params=pltpu.CompilerParams(
            vmem_limit_bytes=63 * 1024 * 1024,
        ),
    )(hidden_states, combine, W_gate, W_up, W_down)
